# Initial kernel scaffold; baseline (speedup 1.0000x reference)
import jax
import jax.numpy as jnp
from jax.experimental import pallas as pl


def kernel(feat, edge_index, edge_rel, W0, b0, Wr, br, Wa):
    raise NotImplementedError("write your pallas kernel here")


# speedup vs baseline: 6.3177x; 6.3177x over previous
import functools

import jax
import jax.numpy as jnp
from jax import lax
from jax.experimental import pallas as pl
from jax.experimental.pallas import tpu as pltpu
from jax.experimental.pallas import tpu_sc as plsc

N = 10000
E = 320000
IN = 128
HID = 128
R = 4

NC = 2
NS = 16
L = 16
B = 128
EPS = 20480
E_PAD = NS * EPS
CHUNKS = EPS // B
ROWS_C = 5120
NA = ROWS_C + B
RPT = NA // NS


def _mm_body(feat_ref, w0_ref, b0_ref, wr_ref, br_ref, selfh_ref, hproj_ref):
    f = feat_ref[...]
    selfh_ref[...] = jnp.dot(f, w0_ref[...],
                             preferred_element_type=jnp.float32) + b0_ref[...]
    for r in range(R):
        hproj_ref[r] = jnp.dot(f, wr_ref[r],
                               preferred_element_type=jnp.float32) + br_ref[r]


def _matmuls(feat, W0, b0r, Wr, brr):
    BN = 512
    grid = (pl.cdiv(N, BN),)
    return pl.pallas_call(
        _mm_body,
        grid=grid,
        in_specs=[
            pl.BlockSpec((BN, IN), lambda i: (i, 0)),
            pl.BlockSpec((IN, HID), lambda i: (0, 0)),
            pl.BlockSpec((1, HID), lambda i: (0, 0)),
            pl.BlockSpec((R, IN, HID), lambda i: (0, 0, 0)),
            pl.BlockSpec((R, 1, HID), lambda i: (0, 0, 0)),
        ],
        out_specs=[
            pl.BlockSpec((BN, HID), lambda i: (i, 0)),
            pl.BlockSpec((R, BN, HID), lambda i: (0, i, 0)),
        ],
        out_shape=[
            jax.ShapeDtypeStruct((N, HID), jnp.float32),
            jax.ShapeDtypeStruct((R, N, HID), jnp.float32),
        ],
    )(feat, W0, b0r, Wr, brr)


def _sc_body(hproj_hbm, src_hbm, rel_hbm, dst_hbm, part_hbm, degp_hbm,
             srcv, relv, dstv, rows, zbuf, deg_local, accum):
    cid = lax.axis_index("c")
    sid = lax.axis_index("s")
    zeros16 = jnp.zeros((L,), jnp.float32)
    base = cid * ROWS_C

    def _fill(i, _):
        for k in range(HID // L):
            zbuf[i, pl.ds(k * L, L)] = zeros16
        return _
    lax.fori_loop(0, 8, _fill, None)

    def _zero(c, _):
        pltpu.sync_copy(zbuf, accum.at[pl.ds(sid * RPT + c * 8, 8)])
        return _
    lax.fori_loop(0, RPT // 8, _zero, None)

    def _zdeg(c, _):
        deg_local[pl.ds(c * L, L)] = zeros16
        return _
    lax.fori_loop(0, NA // L, _zdeg, None)

    pltpu.sync_copy(src_hbm.at[pl.ds(sid * CHUNKS, CHUNKS)], srcv)
    pltpu.sync_copy(rel_hbm.at[pl.ds(sid * CHUNKS, CHUNKS)], relv)
    pltpu.sync_copy(dst_hbm.at[pl.ds(sid * CHUNKS, CHUNKS)], dstv)

    def _mkidx(i, _):
        for j in range(B // L):
            s = srcv[i, pl.ds(j * L, L)]
            r = relv[i, pl.ds(j * L, L)]
            srcv[i, pl.ds(j * L, L)] = r * N + s
            d = dstv[i, pl.ds(j * L, L)] - base
            dl = jnp.where((d >= 0) & (d < ROWS_C), d, ROWS_C)
            dstv[i, pl.ds(j * L, L)] = dl
        return _
    lax.fori_loop(0, CHUNKS, _mkidx, None)

    plsc.subcore_barrier()

    def _chunk(i, _):
        pltpu.sync_copy(hproj_hbm.at[srcv.at[i]], rows)
        pltpu.sync_copy(rows, accum.at[dstv.at[i]], add=True)
        for j in range(B // L):
            dl = dstv[i, pl.ds(j * L, L)]
            cnt, last = plsc.scan_count(dl)
            plsc.addupdate_scatter(deg_local, [dl],
                                   cnt.astype(jnp.float32), mask=last)
        return _
    lax.fori_loop(0, CHUNKS, _chunk, None)

    pltpu.sync_copy(deg_local, degp_hbm.at[cid, sid])
    plsc.subcore_barrier()

    def _out(c, _):
        r0 = sid * RPT + c * 8
        pltpu.sync_copy(accum.at[pl.ds(r0, 8)], zbuf)
        pltpu.sync_copy(zbuf, part_hbm.at[cid, pl.ds(r0, 8)])
        return _
    lax.fori_loop(0, RPT // 8, _out, None)


_sc_call = functools.partial(
    pl.kernel,
    out_type=(
        jax.ShapeDtypeStruct((NC, NA, HID), jnp.float32),
        jax.ShapeDtypeStruct((NC, NS, NA), jnp.float32),
    ),
    mesh=plsc.VectorSubcoreMesh(core_axis_name="c", subcore_axis_name="s"),
    compiler_params=pltpu.CompilerParams(needs_layout_passes=False),
    scratch_types=[
        pltpu.VMEM((CHUNKS, B), jnp.int32),
        pltpu.VMEM((CHUNKS, B), jnp.int32),
        pltpu.VMEM((CHUNKS, B), jnp.int32),
        pltpu.VMEM((B, HID), jnp.float32),
        pltpu.VMEM((8, HID), jnp.float32),
        pltpu.VMEM((NA,), jnp.float32),
        pltpu.VMEM_SHARED((NA, HID), jnp.float32),
    ],
)(_sc_body)


BN_EP = 512
BPC = ROWS_C // BN_EP


def _ep_body(selfh_ref, part_ref, deg_ref, out_ref):
    p = part_ref[0]
    deg = jnp.maximum(jnp.sum(deg_ref[0], axis=0), 1.0)
    mean = p / deg
    out_ref[...] = jnp.maximum(
        jnp.concatenate([selfh_ref[...], mean], axis=1), 0.0)


def _epilogue(self_h, partial, degp):
    grid = (pl.cdiv(N, BN_EP),)
    return pl.pallas_call(
        _ep_body,
        grid=grid,
        in_specs=[
            pl.BlockSpec((BN_EP, HID), lambda i: (i, 0)),
            pl.BlockSpec((1, BN_EP, HID), lambda i: (i // BPC, i % BPC, 0)),
            pl.BlockSpec((1, NS, BN_EP, 1), lambda i: (i // BPC, 0, i % BPC, 0)),
        ],
        out_specs=pl.BlockSpec((BN_EP, 2 * HID), lambda i: (i, 0)),
        out_shape=jax.ShapeDtypeStruct((N, 2 * HID), jnp.float32),
    )(self_h, partial, degp)


def kernel(feat, edge_index, edge_rel, W0, b0, Wr, br, Wa):
    del Wa
    b0r = b0.reshape(1, HID)
    brr = br.reshape(R, 1, HID)

    self_h, hproj = _matmuls(feat, W0, b0r, Wr, brr)
    hproj_flat = hproj.reshape(R * N, HID)

    pad = E_PAD - E
    src = jnp.concatenate([edge_index[0], jnp.zeros((pad,), jnp.int32)])
    rel = jnp.concatenate([edge_rel, jnp.zeros((pad,), jnp.int32)])
    dst = jnp.concatenate([edge_index[1], jnp.full((pad,), N, jnp.int32)])
    src2 = src.reshape(E_PAD // B, B)
    rel2 = rel.reshape(E_PAD // B, B)
    dst2 = dst.reshape(E_PAD // B, B)

    partial, degp = _sc_call(hproj_flat, src2, rel2, dst2)
    return _epilogue(self_h, partial, degp.reshape(NC, NS, NA, 1))


# speedup vs baseline: 6.3668x; 1.0078x over previous
import functools

import jax
import jax.numpy as jnp
from jax import lax
from jax.experimental import pallas as pl
from jax.experimental.pallas import tpu as pltpu
from jax.experimental.pallas import tpu_sc as plsc

N = 10000
E = 320000
IN = 128
HID = 128
R = 4

NC = 2
NS = 16
L = 16
B = 128
EPS = 20480
E_PAD = NS * EPS
CHUNKS = EPS // B
ROWS_C = 5120
NA = ROWS_C + 128
RPT = NA // NS


def _mm_body(feat_ref, w0_ref, b0_ref, wr_ref, br_ref, selfh_ref, hproj_ref):
    f = feat_ref[...]
    selfh_ref[...] = jnp.dot(f, w0_ref[...],
                             preferred_element_type=jnp.float32) + b0_ref[...]
    for r in range(R):
        hproj_ref[r] = jnp.dot(f, wr_ref[r],
                               preferred_element_type=jnp.float32) + br_ref[r]


def _matmuls(feat, W0, b0r, Wr, brr):
    BN = 512
    grid = (pl.cdiv(N, BN),)
    return pl.pallas_call(
        _mm_body,
        grid=grid,
        in_specs=[
            pl.BlockSpec((BN, IN), lambda i: (i, 0)),
            pl.BlockSpec((IN, HID), lambda i: (0, 0)),
            pl.BlockSpec((1, HID), lambda i: (0, 0)),
            pl.BlockSpec((R, IN, HID), lambda i: (0, 0, 0)),
            pl.BlockSpec((R, 1, HID), lambda i: (0, 0, 0)),
        ],
        out_specs=[
            pl.BlockSpec((BN, HID), lambda i: (i, 0)),
            pl.BlockSpec((R, BN, HID), lambda i: (0, i, 0)),
        ],
        out_shape=[
            jax.ShapeDtypeStruct((N, HID), jnp.float32),
            jax.ShapeDtypeStruct((R, N, HID), jnp.float32),
        ],
    )(feat, W0, b0r, Wr, brr)


SLAB = 1
ZSZ = (80, 80, 80, 88)
ZOFF = (0, 80, 160, 240)


def _sc_body(hproj_hbm, src_hbm, rel_hbm, dst_hbm, part_hbm, degp_hbm,
             srcv, relv, rows, zbuf, deg_local, accum):
    cid = lax.axis_index("c")
    sid = lax.axis_index("s")
    zeros16 = jnp.zeros((L,), jnp.float32)
    base = cid * ROWS_C

    def _fill(i, _):
        for k in range(HID // L):
            zbuf[i, pl.ds(k * L, L)] = zeros16
        return _
    lax.fori_loop(0, max(ZSZ), _fill, None)

    for off, sz in zip(ZOFF, ZSZ):
        pltpu.sync_copy(zbuf.at[pl.ds(0, sz)],
                        accum.at[pl.ds(sid * RPT + off, sz)])

    def _zdeg(c, _):
        deg_local[pl.ds(c * L, L)] = zeros16
        return _
    lax.fori_loop(0, NA // L, _zdeg, None)

    pltpu.sync_copy(src_hbm.at[pl.ds(sid * CHUNKS, CHUNKS)], srcv)
    pltpu.sync_copy(rel_hbm.at[pl.ds(sid * CHUNKS, CHUNKS)], relv)

    def _mkidx(i, _):
        for j in range(B // L):
            s = srcv[i, pl.ds(j * L, L)]
            r = relv[i, pl.ds(j * L, L)]
            srcv[i, pl.ds(j * L, L)] = r * N + s
        return _
    lax.fori_loop(0, CHUNKS, _mkidx, None)

    pltpu.sync_copy(dst_hbm.at[pl.ds(sid * CHUNKS, CHUNKS)], relv)

    def _mkdst(i, _):
        for j in range(B // L):
            d = relv[i, pl.ds(j * L, L)] - base
            dl = jnp.where((d >= 0) & (d < ROWS_C), d, ROWS_C)
            relv[i, pl.ds(j * L, L)] = dl
        return _
    lax.fori_loop(0, CHUNKS, _mkdst, None)

    plsc.subcore_barrier()

    def _grp(g, _):
        pltpu.sync_copy(hproj_hbm.at[srcv.at[g]], rows)
        pltpu.sync_copy(rows, accum.at[relv.at[g]], add=True)
        for t in range(SLAB):
            for j in range(B // L):
                dl = relv[g * SLAB + t, pl.ds(j * L, L)]
                cnt, last = plsc.scan_count(dl)
                plsc.addupdate_scatter(deg_local, [dl],
                                       cnt.astype(jnp.float32), mask=last)
        return _
    lax.fori_loop(0, CHUNKS // SLAB, _grp, None)

    pltpu.sync_copy(deg_local, degp_hbm.at[cid, sid])

    plsc.subcore_barrier()

    for off, sz in zip(ZOFF, ZSZ):
        r0 = sid * RPT + off
        pltpu.sync_copy(accum.at[pl.ds(r0, sz)], zbuf.at[pl.ds(0, sz)])
        pltpu.sync_copy(zbuf.at[pl.ds(0, sz)],
                        part_hbm.at[cid, pl.ds(r0, sz)])


_sc_call = functools.partial(
    pl.kernel,
    out_type=(
        jax.ShapeDtypeStruct((NC, NA, HID), jnp.float32),
        jax.ShapeDtypeStruct((NC, NS, NA), jnp.float32),
    ),
    mesh=plsc.VectorSubcoreMesh(core_axis_name="c", subcore_axis_name="s"),
    compiler_params=pltpu.CompilerParams(needs_layout_passes=False),
    scratch_types=[
        pltpu.VMEM((CHUNKS, B), jnp.int32),
        pltpu.VMEM((CHUNKS, B), jnp.int32),
        pltpu.VMEM((SLAB * B, HID), jnp.float32),
        pltpu.VMEM((max(ZSZ), HID), jnp.float32),
        pltpu.VMEM((NA,), jnp.float32),
        pltpu.VMEM_SHARED((NA, HID), jnp.float32),
    ],
)(_sc_body)


BN_EP = 512
BPC = ROWS_C // BN_EP


def _ep_body(selfh_ref, part_ref, deg_ref, out_ref):
    p = part_ref[0]
    deg = jnp.maximum(jnp.sum(deg_ref[0], axis=0), 1.0)
    mean = p / deg
    out_ref[...] = jnp.maximum(
        jnp.concatenate([selfh_ref[...], mean], axis=1), 0.0)


def _epilogue(self_h, partial, degp):
    grid = (pl.cdiv(N, BN_EP),)
    return pl.pallas_call(
        _ep_body,
        grid=grid,
        in_specs=[
            pl.BlockSpec((BN_EP, HID), lambda i: (i, 0)),
            pl.BlockSpec((1, BN_EP, HID), lambda i: (i // BPC, i % BPC, 0)),
            pl.BlockSpec((1, NS, BN_EP, 1), lambda i: (i // BPC, 0, i % BPC, 0)),
        ],
        out_specs=pl.BlockSpec((BN_EP, 2 * HID), lambda i: (i, 0)),
        out_shape=jax.ShapeDtypeStruct((N, 2 * HID), jnp.float32),
    )(self_h, partial, degp)


def kernel(feat, edge_index, edge_rel, W0, b0, Wr, br, Wa):
    del Wa
    b0r = b0.reshape(1, HID)
    brr = br.reshape(R, 1, HID)

    self_h, hproj = _matmuls(feat, W0, b0r, Wr, brr)
    hproj_flat = hproj.reshape(R * N, HID)

    pad = E_PAD - E
    src = jnp.concatenate([edge_index[0], jnp.zeros((pad,), jnp.int32)])
    rel = jnp.concatenate([edge_rel, jnp.zeros((pad,), jnp.int32)])
    dst = jnp.concatenate([edge_index[1], jnp.full((pad,), N, jnp.int32)])
    src2 = src.reshape(E_PAD // B, B)
    rel2 = rel.reshape(E_PAD // B, B)
    dst2 = dst.reshape(E_PAD // B, B)

    partial, degp = _sc_call(hproj_flat, src2, rel2, dst2)
    return _epilogue(self_h, partial, degp.reshape(NC, NS, NA, 1))
